# baseline (speedup 1.0000x reference)
import functools

import jax
import jax.numpy as jnp
from jax import lax
from jax.experimental import pallas as pl
from jax.experimental.pallas import tpu as pltpu
from jax.experimental.pallas import tpu_sc as plsc

N = 10000
E = 320000
D = 128

NC = 2
NS = 16
NW = NC * NS
EPW = E // NW
CH = 80
NCH = EPW // CH
STRIPE = 624
TAIL0 = N - NS * STRIPE
SR = 16
CROWS = 1280
CSTRIPE = CROWS // NS

_SQRT_HALF = 0.7071067811865476


def _gelu(x):
    return 0.5 * x * (1.0 + lax.erf(x * _SQRT_HALF))


def _node_pre_body(f_ref, p_ref, wa1, ba1, wa2, ba2, wea, web, wec,
                   s_ref, t_ref):
    f = f_ref[...]
    p = p_ref[...]
    h = _gelu(jnp.dot(f, wa1[...], preferred_element_type=jnp.float32)
              + ba1[...])
    off = _gelu(jnp.dot(h, wa2[...], preferred_element_type=jnp.float32)
                + ba2[...])
    adj = p + off
    s_ref[...] = (jnp.dot(f, wea[...], preferred_element_type=jnp.float32)
                  + jnp.dot(adj, wec[...], preferred_element_type=jnp.float32))
    t_ref[...] = (jnp.dot(f, web[...], preferred_element_type=jnp.float32)
                  - jnp.dot(p, wec[...], preferred_element_type=jnp.float32))


def _node_precompute(features, points8, Wa1, ba1, Wa2, ba2, Wea, Web, Wec):
    nb = 2000
    grid = N // nb
    full = lambda shape: pl.BlockSpec(shape, lambda i: (0, 0))
    return pl.pallas_call(
        _node_pre_body,
        grid=(grid,),
        in_specs=[
            pl.BlockSpec((nb, D), lambda i: (i, 0)),
            pl.BlockSpec((nb, 8), lambda i: (i, 0)),
            full((D, 64)), full((1, 64)), full((64, 8)), full((1, 8)),
            full((D, D)), full((D, D)), full((8, D)),
        ],
        out_specs=[
            pl.BlockSpec((nb, D), lambda i: (i, 0)),
            pl.BlockSpec((nb, D), lambda i: (i, 0)),
        ],
        out_shape=[
            jax.ShapeDtypeStruct((N, D), jnp.float32),
            jax.ShapeDtypeStruct((N, D), jnp.float32),
        ],
    )(features, points8, Wa1, ba1, Wa2, ba2, Wea, Web, Wec)


def _sc_gather(S, T, src, dst, zeros_d, pat):
    e_half = src.shape[0]
    epw = e_half // NW
    nch = epw // CH
    npairs = nch // 2
    odd = nch % 2 == 1
    mesh = plsc.VectorSubcoreMesh(core_axis_name="c", subcore_axis_name="s")

    @functools.partial(
        pl.kernel, mesh=mesh,
        out_type=[
            jax.ShapeDtypeStruct((e_half, D), jnp.float32),
            jax.ShapeDtypeStruct((e_half, D), jnp.float32),
            jax.ShapeDtypeStruct((NC, CROWS, D), jnp.float32),
        ],
        scratch_types=[
            pltpu.VMEM((CH,), jnp.int32),
            pltpu.VMEM((CH,), jnp.int32),
            pltpu.VMEM((CH,), jnp.int32),
            pltpu.VMEM((CH,), jnp.int32),
            pltpu.VMEM((CH,), jnp.int32),
            pltpu.VMEM((CH,), jnp.int32),
            pltpu.VMEM((CH,), jnp.int32),
            pltpu.VMEM((CH,), jnp.int32),
            pltpu.VMEM((CH, D), jnp.float32),
            pltpu.VMEM((CH, D), jnp.float32),
            pltpu.VMEM((CH, D), jnp.float32),
            pltpu.VMEM((CH, D), jnp.float32),
            pltpu.VMEM((CH, D), jnp.float32),
            pltpu.VMEM((CH, D), jnp.float32),
            pltpu.VMEM((8, D), jnp.float32),
            pltpu.SemaphoreType.DMA,
            pltpu.SemaphoreType.DMA,
            pltpu.SemaphoreType.DMA,
            pltpu.SemaphoreType.DMA,
            pltpu.SemaphoreType.DMA,
            pltpu.SemaphoreType.DMA,
            pltpu.SemaphoreType.DMA,
            pltpu.SemaphoreType.DMA,
            pltpu.SemaphoreType.DMA,
            pltpu.SemaphoreType.DMA,
            pltpu.SemaphoreType.DMA,
            pltpu.VMEM_SHARED((CROWS, D), jnp.float32),
            pltpu.VMEM_SHARED((8, D), jnp.float32),
        ],
    )
    def k(s_hbm, t_hbm, src_hbm, dst_hbm, zd_hbm, pat_hbm,
          gs_hbm, gt_hbm, cnt_hbm,
          is0, id0, is1, id1, im0, ih0, im1, ih1,
          bs0, bt0, bs1, bt1, pbuf0, pbuf1, patv,
          si0, si1, sg0, sg1, sw0, sw1, sp0, sp1, sa0, sa1, spv,
          cacc, pat_sp):
        c = lax.axis_index("c")
        s = lax.axis_index("s")
        base = (s * NC + c) * epw

        pltpu.sync_copy(zd_hbm.at[pl.ds(0, CSTRIPE)], bs0)
        pltpu.sync_copy(bs0, cacc.at[pl.ds(s * CSTRIPE, CSTRIPE)])

        @pl.when(s == 0)
        def _initp():
            pltpu.sync_copy(pat_hbm, patv)
            pltpu.sync_copy(patv, pat_sp)
        plsc.subcore_barrier()

        slots = ((is0, id0, im0, ih0, bs0, bt0, si0, sg0, sw0, pbuf0, sp0, sa0),
                 (is1, id1, im1, ih1, bs1, bt1, si1, sg1, sw1, pbuf1, sp1, sa1))

        def start_idx(g, sl):
            o = base + g * CH
            pltpu.async_copy(src_hbm.at[pl.ds(o, CH)], sl[0], sl[6])
            pltpu.async_copy(dst_hbm.at[pl.ds(o, CH)], sl[1], sl[6])

        def wait_idx(sl):
            pltpu.make_async_copy(src_hbm.at[pl.ds(0, CH)], sl[0],
                                  sl[6]).wait()
            pltpu.make_async_copy(dst_hbm.at[pl.ds(0, CH)], sl[1],
                                  sl[6]).wait()

        def start_gather(sl):
            pltpu.async_copy(s_hbm.at[sl[0]], sl[4], sl[7])
            pltpu.async_copy(t_hbm.at[sl[1]], sl[5], sl[7])

        def wait_gather(sl):
            pltpu.make_async_copy(s_hbm.at[sl[0]], sl[4], sl[7]).wait()
            pltpu.make_async_copy(t_hbm.at[sl[1]], sl[5], sl[7]).wait()

        def start_write(g, sl):
            o = base + g * CH
            pltpu.async_copy(sl[4], gs_hbm.at[pl.ds(o, CH)], sl[8])
            pltpu.async_copy(sl[5], gt_hbm.at[pl.ds(o, CH)], sl[8])

        def wait_write(sl):
            pltpu.make_async_copy(sl[4], gs_hbm.at[pl.ds(0, CH)],
                                  sl[8]).wait()
            pltpu.make_async_copy(sl[5], gt_hbm.at[pl.ds(0, CH)],
                                  sl[8]).wait()

        def counts_start(sl):
            for j in range(CH // 16):
                v = sl[1][pl.ds(j * 16, 16)]
                sl[2][pl.ds(j * 16, 16)] = jnp.bitwise_and(v, 7)
                sl[3][pl.ds(j * 16, 16)] = lax.shift_right_logical(v, 3)
            pltpu.async_copy(pat_sp.at[sl[2]], sl[9], sl[10])

        def counts_add(sl):
            pltpu.make_async_copy(pat_sp.at[sl[2]], sl[9], sl[10]).wait()
            pltpu.async_copy(sl[9], cacc.at[sl[3]], sl[11], add=True)

        def counts_drain(sl):
            pltpu.make_async_copy(sl[9], cacc.at[sl[3]], sl[11]).wait()

        start_idx(0, slots[0])
        start_idx(1, slots[1])

        def body(i, _):
            for b in (0, 1):
                sl = slots[b]
                wait_idx(sl)

                @pl.when(i > 0)
                def _w(sl=sl):
                    wait_write(sl)
                start_gather(sl)
            for b in (0, 1):
                sl = slots[b]
                g = 2 * i + b
                wait_gather(sl)
                start_write(g, sl)

                @pl.when(i > 0)
                def _d(sl=sl):
                    counts_drain(sl)
                counts_start(sl)

                @pl.when(g + 2 < nch)
                def _n(g=g, sl=sl):
                    start_idx(g + 2, sl)
                counts_add(sl)
            return ()

        lax.fori_loop(0, npairs, body, ())
        if odd:
            sl = slots[0]
            wait_idx(sl)
            wait_write(sl)
            start_gather(sl)
            wait_gather(sl)
            start_write(nch - 1, sl)
            counts_drain(sl)
            counts_start(sl)
            counts_add(sl)
            counts_drain(sl)
            counts_drain(slots[1])
            wait_write(sl)
            wait_write(slots[1])
        else:
            counts_drain(slots[0])
            counts_drain(slots[1])
            wait_write(slots[0])
            wait_write(slots[1])

        plsc.subcore_barrier()
        pltpu.sync_copy(cacc.at[pl.ds(s * CSTRIPE, CSTRIPE)], bs1)
        pltpu.sync_copy(bs1, cnt_hbm.at[c, pl.ds(s * CSTRIPE, CSTRIPE)])

    return k(S, T, src, dst, zeros_d, pat)


def _edge_mlp_body(gs_ref, gt_ref, be1, we2, be2, h2_ref):
    x = gs_ref[...] + gt_ref[...] + be1[...]
    h = _gelu(x)
    h2_ref[...] = _gelu(
        jnp.dot(h, we2[...], preferred_element_type=jnp.float32) + be2[...])


def _edge_mlp(gs, gt, be1, We2, be2):
    eb = 2560
    ne = gs.shape[0]
    grid = ne // eb
    full = lambda shape: pl.BlockSpec(shape, lambda i: (0, 0))
    return pl.pallas_call(
        _edge_mlp_body,
        grid=(grid,),
        in_specs=[
            pl.BlockSpec((eb, D), lambda i: (i, 0)),
            pl.BlockSpec((eb, D), lambda i: (i, 0)),
            full((1, D)), full((D, D)), full((1, D)),
        ],
        out_specs=pl.BlockSpec((eb, D), lambda i: (i, 0)),
        out_shape=jax.ShapeDtypeStruct((ne, D), jnp.float32),
    )(gs, gt, be1, We2, be2)


def _sc_scatter(h2, dst, zeros_d):
    epw = h2.shape[0] // NW
    mesh = plsc.VectorSubcoreMesh(core_axis_name="c", subcore_axis_name="s")

    @functools.partial(
        pl.kernel, mesh=mesh,
        out_type=[
            jax.ShapeDtypeStruct((NC, N, D), jnp.float32),
        ],
        scratch_types=[
            pltpu.VMEM((CH,), jnp.int32),
            pltpu.VMEM((CH,), jnp.int32),
            pltpu.VMEM((CH, D), jnp.float32),
            pltpu.VMEM((CH, D), jnp.float32),
            pltpu.VMEM((SR, D), jnp.float32),
            pltpu.SemaphoreType.DMA,
            pltpu.SemaphoreType.DMA,
            pltpu.VMEM_SHARED((N, D), jnp.float32),
        ],
    )
    def k(h2_hbm, dst_hbm, zd_hbm, sum_hbm,
          idx0, idx1, b0, b1, stage, sl0, sl1, acc):
        c = lax.axis_index("c")
        s = lax.axis_index("s")
        r0 = s * STRIPE
        pltpu.sync_copy(zd_hbm.at[pl.ds(0, SR)], stage)
        for kk in range(STRIPE // SR):
            pltpu.sync_copy(stage, acc.at[pl.ds(r0 + kk * SR, SR)])

        @pl.when(s == 0)
        def _zero_tail():
            pltpu.sync_copy(stage.at[pl.ds(0, TAIL0)],
                            acc.at[pl.ds(NS * STRIPE, TAIL0)])
        plsc.subcore_barrier()

        slots = ((idx0, b0, sl0), (idx1, b1, sl1))
        base = (c * NS + s) * epw
        nch = epw // CH
        npairs = nch // 2
        odd = nch % 2 == 1

        def start_load(g, sl):
            o = base + g * CH
            pltpu.async_copy(dst_hbm.at[pl.ds(o, CH)], sl[0], sl[2])
            pltpu.async_copy(h2_hbm.at[pl.ds(o, CH)], sl[1], sl[2])

        def wait_load(sl):
            pltpu.make_async_copy(dst_hbm.at[pl.ds(0, CH)], sl[0],
                                  sl[2]).wait()
            pltpu.make_async_copy(h2_hbm.at[pl.ds(0, CH)], sl[1],
                                  sl[2]).wait()

        start_load(0, slots[0])
        start_load(1, slots[1])

        def body(i, _):
            for b in (0, 1):
                sl = slots[b]
                g = 2 * i + b
                wait_load(sl)
                pltpu.sync_copy(sl[1], acc.at[sl[0]], add=True)

                @pl.when(g + 2 < nch)
                def _n(g=g, sl=sl):
                    start_load(g + 2, sl)
            return ()

        lax.fori_loop(0, npairs, body, ())
        if odd:
            sl = slots[0]
            wait_load(sl)
            pltpu.sync_copy(sl[1], acc.at[sl[0]], add=True)

        plsc.subcore_barrier()
        for kk in range(STRIPE // SR):
            pltpu.sync_copy(acc.at[pl.ds(r0 + kk * SR, SR)], stage)
            pltpu.sync_copy(stage, sum_hbm.at[c, pl.ds(r0 + kk * SR, SR)])

        @pl.when(s == 0)
        def _out_tail():
            pltpu.sync_copy(acc.at[pl.ds(NS * STRIPE, TAIL0)],
                            stage.at[pl.ds(0, TAIL0)])
            pltpu.sync_copy(stage.at[pl.ds(0, TAIL0)],
                            sum_hbm.at[c, pl.ds(NS * STRIPE, TAIL0)])

    return k(h2, dst, zeros_d)


def _final_body(suma_ref, sumb_ref, cnt_ref, f_ref, wo1, bo1, wo2, bo2,
                out_ref):
    ssum = suma_ref[0] + suma_ref[1] + sumb_ref[0] + sumb_ref[1]
    csum = cnt_ref[0] + cnt_ref[1]
    agg = ssum / jnp.maximum(csum, 1.0)
    u = _gelu(jnp.dot(agg, wo1[...], preferred_element_type=jnp.float32)
              + bo1[...])
    out_ref[...] = (_gelu(jnp.dot(u, wo2[...],
                                  preferred_element_type=jnp.float32)
                          + bo2[...]) + f_ref[...])


def _final_mlp(sums_a, sums_b, cnts, features, Wo1, bo1, Wo2, bo2):
    nb = 2000
    grid = N // nb
    full = lambda shape: pl.BlockSpec(shape, lambda i: tuple(0 for _ in shape))
    return pl.pallas_call(
        _final_body,
        grid=(grid,),
        in_specs=[
            pl.BlockSpec((NC, nb, D), lambda i: (0, i, 0)),
            pl.BlockSpec((NC, nb, D), lambda i: (0, i, 0)),
            pl.BlockSpec((NC, nb, 1), lambda i: (0, i, 0)),
            pl.BlockSpec((nb, D), lambda i: (i, 0)),
            full((D, D)), full((1, D)), full((D, D)), full((1, D)),
        ],
        out_specs=pl.BlockSpec((nb, D), lambda i: (i, 0)),
        out_shape=jax.ShapeDtypeStruct((N, D), jnp.float32),
    )(sums_a, sums_b, cnts, features, Wo1, bo1, Wo2, bo2)


def kernel(features, points, l0_edges, Wa1, ba1, Wa2, ba2,
           We1, be1, We2, be2, Wo1, bo1, Wo2, bo2):
    src = l0_edges[:, 0]
    dst = l0_edges[:, 1]

    points8 = jnp.pad(points, ((0, 0), (0, 5)))
    Wa2p = jnp.pad(Wa2, ((0, 0), (0, 5)))
    ba2p = jnp.pad(ba2, ((0, 5))).reshape(1, 8)
    Wea = We1[:D]
    Web = We1[D:2 * D]
    Wec = jnp.pad(We1[2 * D:], ((0, 5), (0, 0)))

    S, T = _node_precompute(features, points8, Wa1, ba1.reshape(1, 64),
                            Wa2p, ba2p, Wea, Web, Wec)
    zeros_d = jnp.zeros((N, D), jnp.float32)
    pat = jnp.repeat(jnp.eye(8, dtype=jnp.float32), 16, axis=1)
    EA = 161280
    srca, dsta = src[:EA], dst[:EA]
    srcb, dstb = src[EA:], dst[EA:]
    gsa, gta, cnta = _sc_gather(S, T, srca, dsta, zeros_d, pat)
    gsb, gtb, cntb = _sc_gather(S, T, srcb, dstb, zeros_d, pat)
    be1r, be2r = be1.reshape(1, D), be2.reshape(1, D)
    h2a = _edge_mlp(gsa, gta, be1r, We2, be2r)
    h2b = _edge_mlp(gsb, gtb, be1r, We2, be2r)
    (sums_a,) = _sc_scatter(h2a, dsta, zeros_d)
    (sums_b,) = _sc_scatter(h2b, dstb, zeros_d)
    cnts_packed = cnta + cntb
    cnts = cnts_packed[:, :N // 8, :].reshape(NC, N // 8, 8, 16)[..., 0]
    cnts = cnts.reshape(NC, N, 1)
    return _final_mlp(sums_a, sums_b, cnts, features, Wo1, bo1.reshape(1, D),
                      Wo2, bo2.reshape(1, D))


# speedup vs baseline: 1.0223x; 1.0223x over previous
import functools

import jax
import jax.numpy as jnp
from jax import lax
from jax.experimental import pallas as pl
from jax.experimental.pallas import tpu as pltpu
from jax.experimental.pallas import tpu_sc as plsc

N = 10000
E = 320000
D = 128

NC = 2
NS = 16
NW = NC * NS
EPW = E // NW
CH = 80
NCH = EPW // CH
STRIPE = 624
TAIL0 = N - NS * STRIPE
SR = 16
CROWS = 1280
CSTRIPE = CROWS // NS

_SQRT_HALF = 0.7071067811865476


def _gelu(x):
    return 0.5 * x * (1.0 + lax.erf(x * _SQRT_HALF))


def _node_pre_body(f_ref, p_ref, wa1, ba1, wa2, ba2, wea, web, wec,
                   s_ref, t_ref):
    f = f_ref[...]
    p = p_ref[...]
    h = _gelu(jnp.dot(f, wa1[...], preferred_element_type=jnp.float32)
              + ba1[...])
    off = _gelu(jnp.dot(h, wa2[...], preferred_element_type=jnp.float32)
                + ba2[...])
    adj = p + off
    s_ref[...] = (jnp.dot(f, wea[...], preferred_element_type=jnp.float32)
                  + jnp.dot(adj, wec[...], preferred_element_type=jnp.float32))
    t_ref[...] = (jnp.dot(f, web[...], preferred_element_type=jnp.float32)
                  - jnp.dot(p, wec[...], preferred_element_type=jnp.float32))


def _node_precompute(features, points8, Wa1, ba1, Wa2, ba2, Wea, Web, Wec):
    nb = 2000
    grid = N // nb
    full = lambda shape: pl.BlockSpec(shape, lambda i: (0, 0))
    return pl.pallas_call(
        _node_pre_body,
        grid=(grid,),
        in_specs=[
            pl.BlockSpec((nb, D), lambda i: (i, 0)),
            pl.BlockSpec((nb, 8), lambda i: (i, 0)),
            full((D, 64)), full((1, 64)), full((64, 8)), full((1, 8)),
            full((D, D)), full((D, D)), full((8, D)),
        ],
        out_specs=[
            pl.BlockSpec((nb, D), lambda i: (i, 0)),
            pl.BlockSpec((nb, D), lambda i: (i, 0)),
        ],
        out_shape=[
            jax.ShapeDtypeStruct((N, D), jnp.float32),
            jax.ShapeDtypeStruct((N, D), jnp.float32),
        ],
    )(features, points8, Wa1, ba1, Wa2, ba2, Wea, Web, Wec)


def _sc_gather(S, T, src, dst, zeros_d, pat):
    e_half = src.shape[0]
    epw = e_half // NW
    nch = epw // CH
    npairs = nch // 2
    odd = nch % 2 == 1
    mesh = plsc.VectorSubcoreMesh(core_axis_name="c", subcore_axis_name="s")

    @functools.partial(
        pl.kernel, mesh=mesh,
        out_type=[
            jax.ShapeDtypeStruct((e_half, D), jnp.float32),
            jax.ShapeDtypeStruct((e_half, D), jnp.float32),
            jax.ShapeDtypeStruct((NC, CROWS, D), jnp.float32),
        ],
        scratch_types=[
            pltpu.VMEM((CH,), jnp.int32),
            pltpu.VMEM((CH,), jnp.int32),
            pltpu.VMEM((CH,), jnp.int32),
            pltpu.VMEM((CH,), jnp.int32),
            pltpu.VMEM((CH,), jnp.int32),
            pltpu.VMEM((CH,), jnp.int32),
            pltpu.VMEM((CH,), jnp.int32),
            pltpu.VMEM((CH,), jnp.int32),
            pltpu.VMEM((CH, D), jnp.float32),
            pltpu.VMEM((CH, D), jnp.float32),
            pltpu.VMEM((CH, D), jnp.float32),
            pltpu.VMEM((CH, D), jnp.float32),
            pltpu.VMEM((CH, D), jnp.float32),
            pltpu.VMEM((CH, D), jnp.float32),
            pltpu.VMEM((8, D), jnp.float32),
            pltpu.SemaphoreType.DMA,
            pltpu.SemaphoreType.DMA,
            pltpu.SemaphoreType.DMA,
            pltpu.SemaphoreType.DMA,
            pltpu.SemaphoreType.DMA,
            pltpu.SemaphoreType.DMA,
            pltpu.SemaphoreType.DMA,
            pltpu.SemaphoreType.DMA,
            pltpu.SemaphoreType.DMA,
            pltpu.SemaphoreType.DMA,
            pltpu.SemaphoreType.DMA,
            pltpu.VMEM_SHARED((CROWS, D), jnp.float32),
            pltpu.VMEM_SHARED((8, D), jnp.float32),
        ],
    )
    def k(s_hbm, t_hbm, src_hbm, dst_hbm, zd_hbm, pat_hbm,
          gs_hbm, gt_hbm, cnt_hbm,
          is0, id0, is1, id1, im0, ih0, im1, ih1,
          bs0, bt0, bs1, bt1, pbuf0, pbuf1, patv,
          si0, si1, sg0, sg1, sw0, sw1, sp0, sp1, sa0, sa1, spv,
          cacc, pat_sp):
        c = lax.axis_index("c")
        s = lax.axis_index("s")
        base = (s * NC + c) * epw

        pltpu.sync_copy(zd_hbm.at[pl.ds(0, CSTRIPE)], bs0)
        pltpu.sync_copy(bs0, cacc.at[pl.ds(s * CSTRIPE, CSTRIPE)])

        @pl.when(s == 0)
        def _initp():
            pltpu.sync_copy(pat_hbm, patv)
            pltpu.sync_copy(patv, pat_sp)
        plsc.subcore_barrier()

        slots = ((is0, id0, im0, ih0, bs0, bt0, si0, sg0, sw0, pbuf0, sp0, sa0),
                 (is1, id1, im1, ih1, bs1, bt1, si1, sg1, sw1, pbuf1, sp1, sa1))

        def start_idx(g, sl):
            o = base + g * CH
            pltpu.async_copy(src_hbm.at[pl.ds(o, CH)], sl[0], sl[6])
            pltpu.async_copy(dst_hbm.at[pl.ds(o, CH)], sl[1], sl[6])

        def wait_idx(sl):
            pltpu.make_async_copy(src_hbm.at[pl.ds(0, CH)], sl[0],
                                  sl[6]).wait()
            pltpu.make_async_copy(dst_hbm.at[pl.ds(0, CH)], sl[1],
                                  sl[6]).wait()

        def start_gather(sl):
            pltpu.async_copy(s_hbm.at[sl[0]], sl[4], sl[7])
            pltpu.async_copy(t_hbm.at[sl[1]], sl[5], sl[7])

        def wait_gather(sl):
            pltpu.make_async_copy(s_hbm.at[sl[0]], sl[4], sl[7]).wait()
            pltpu.make_async_copy(t_hbm.at[sl[1]], sl[5], sl[7]).wait()

        def start_write(g, sl):
            o = base + g * CH
            pltpu.async_copy(sl[4], gs_hbm.at[pl.ds(o, CH)], sl[8])
            pltpu.async_copy(sl[5], gt_hbm.at[pl.ds(o, CH)], sl[8])

        def wait_write(sl):
            pltpu.make_async_copy(sl[4], gs_hbm.at[pl.ds(0, CH)],
                                  sl[8]).wait()
            pltpu.make_async_copy(sl[5], gt_hbm.at[pl.ds(0, CH)],
                                  sl[8]).wait()

        def counts_start(sl):
            for j in range(CH // 16):
                v = sl[1][pl.ds(j * 16, 16)]
                sl[2][pl.ds(j * 16, 16)] = jnp.bitwise_and(v, 7)
                sl[3][pl.ds(j * 16, 16)] = lax.shift_right_logical(v, 3)
            pltpu.async_copy(pat_sp.at[sl[2]], sl[9], sl[10])

        def counts_add(sl):
            pltpu.make_async_copy(pat_sp.at[sl[2]], sl[9], sl[10]).wait()
            pltpu.async_copy(sl[9], cacc.at[sl[3]], sl[11], add=True)

        def counts_drain(sl):
            pltpu.make_async_copy(sl[9], cacc.at[sl[3]], sl[11]).wait()

        start_idx(0, slots[0])
        start_idx(1, slots[1])

        def body(i, _):
            for b in (0, 1):
                sl = slots[b]
                wait_idx(sl)

                @pl.when(i > 0)
                def _w(sl=sl):
                    wait_write(sl)
                start_gather(sl)
            for b in (0, 1):
                sl = slots[b]
                g = 2 * i + b
                wait_gather(sl)
                start_write(g, sl)

                @pl.when(i > 0)
                def _d(sl=sl):
                    counts_drain(sl)
                counts_start(sl)

                @pl.when(g + 2 < nch)
                def _n(g=g, sl=sl):
                    start_idx(g + 2, sl)
                counts_add(sl)
            return ()

        lax.fori_loop(0, npairs, body, ())
        if odd:
            sl = slots[0]
            wait_idx(sl)
            wait_write(sl)
            start_gather(sl)
            wait_gather(sl)
            start_write(nch - 1, sl)
            counts_drain(sl)
            counts_start(sl)
            counts_add(sl)
            counts_drain(sl)
            counts_drain(slots[1])
            wait_write(sl)
            wait_write(slots[1])
        else:
            counts_drain(slots[0])
            counts_drain(slots[1])
            wait_write(slots[0])
            wait_write(slots[1])

        plsc.subcore_barrier()
        pltpu.sync_copy(cacc.at[pl.ds(s * CSTRIPE, CSTRIPE)], bs1)
        pltpu.sync_copy(bs1, cnt_hbm.at[c, pl.ds(s * CSTRIPE, CSTRIPE)])

    return k(S, T, src, dst, zeros_d, pat)


def _edge_mlp_body(gs_ref, gt_ref, be1, we2, be2, h2_ref):
    x = gs_ref[...] + gt_ref[...] + be1[...]
    h = _gelu(x)
    h2_ref[...] = _gelu(
        jnp.dot(h, we2[...], preferred_element_type=jnp.float32) + be2[...])


def _edge_mlp(gs, gt, be1, We2, be2):
    eb = 6400
    ne = gs.shape[0]
    grid = ne // eb
    full = lambda shape: pl.BlockSpec(shape, lambda i: (0, 0))
    return pl.pallas_call(
        _edge_mlp_body,
        grid=(grid,),
        in_specs=[
            pl.BlockSpec((eb, D), lambda i: (i, 0)),
            pl.BlockSpec((eb, D), lambda i: (i, 0)),
            full((1, D)), full((D, D)), full((1, D)),
        ],
        out_specs=pl.BlockSpec((eb, D), lambda i: (i, 0)),
        out_shape=jax.ShapeDtypeStruct((ne, D), jnp.float32),
    )(gs, gt, be1, We2, be2)


def _sc_scatter(h2, dst, zeros_d):
    epw = h2.shape[0] // NW
    mesh = plsc.VectorSubcoreMesh(core_axis_name="c", subcore_axis_name="s")

    @functools.partial(
        pl.kernel, mesh=mesh,
        out_type=[
            jax.ShapeDtypeStruct((NC, N, D), jnp.float32),
        ],
        scratch_types=[
            pltpu.VMEM((CH,), jnp.int32),
            pltpu.VMEM((CH,), jnp.int32),
            pltpu.VMEM((CH, D), jnp.float32),
            pltpu.VMEM((CH, D), jnp.float32),
            pltpu.VMEM((SR, D), jnp.float32),
            pltpu.SemaphoreType.DMA,
            pltpu.SemaphoreType.DMA,
            pltpu.VMEM_SHARED((N, D), jnp.float32),
        ],
    )
    def k(h2_hbm, dst_hbm, zd_hbm, sum_hbm,
          idx0, idx1, b0, b1, stage, sl0, sl1, acc):
        c = lax.axis_index("c")
        s = lax.axis_index("s")
        r0 = s * STRIPE
        pltpu.sync_copy(zd_hbm.at[pl.ds(0, SR)], stage)
        for kk in range(STRIPE // SR):
            pltpu.sync_copy(stage, acc.at[pl.ds(r0 + kk * SR, SR)])

        @pl.when(s == 0)
        def _zero_tail():
            pltpu.sync_copy(stage.at[pl.ds(0, TAIL0)],
                            acc.at[pl.ds(NS * STRIPE, TAIL0)])
        plsc.subcore_barrier()

        slots = ((idx0, b0, sl0), (idx1, b1, sl1))
        base = (c * NS + s) * epw
        nch = epw // CH
        npairs = nch // 2
        odd = nch % 2 == 1

        def start_load(g, sl):
            o = base + g * CH
            pltpu.async_copy(dst_hbm.at[pl.ds(o, CH)], sl[0], sl[2])
            pltpu.async_copy(h2_hbm.at[pl.ds(o, CH)], sl[1], sl[2])

        def wait_load(sl):
            pltpu.make_async_copy(dst_hbm.at[pl.ds(0, CH)], sl[0],
                                  sl[2]).wait()
            pltpu.make_async_copy(h2_hbm.at[pl.ds(0, CH)], sl[1],
                                  sl[2]).wait()

        start_load(0, slots[0])
        start_load(1, slots[1])

        def body(i, _):
            for b in (0, 1):
                sl = slots[b]
                g = 2 * i + b
                wait_load(sl)
                pltpu.sync_copy(sl[1], acc.at[sl[0]], add=True)

                @pl.when(g + 2 < nch)
                def _n(g=g, sl=sl):
                    start_load(g + 2, sl)
            return ()

        lax.fori_loop(0, npairs, body, ())
        if odd:
            sl = slots[0]
            wait_load(sl)
            pltpu.sync_copy(sl[1], acc.at[sl[0]], add=True)

        plsc.subcore_barrier()
        for kk in range(STRIPE // SR):
            pltpu.sync_copy(acc.at[pl.ds(r0 + kk * SR, SR)], stage)
            pltpu.sync_copy(stage, sum_hbm.at[c, pl.ds(r0 + kk * SR, SR)])

        @pl.when(s == 0)
        def _out_tail():
            pltpu.sync_copy(acc.at[pl.ds(NS * STRIPE, TAIL0)],
                            stage.at[pl.ds(0, TAIL0)])
            pltpu.sync_copy(stage.at[pl.ds(0, TAIL0)],
                            sum_hbm.at[c, pl.ds(NS * STRIPE, TAIL0)])

    return k(h2, dst, zeros_d)


def _final_body(suma_ref, sumb_ref, cnt_ref, f_ref, wo1, bo1, wo2, bo2,
                out_ref):
    ssum = suma_ref[0] + suma_ref[1] + sumb_ref[0] + sumb_ref[1]
    csum = cnt_ref[0] + cnt_ref[1]
    agg = ssum / jnp.maximum(csum, 1.0)
    u = _gelu(jnp.dot(agg, wo1[...], preferred_element_type=jnp.float32)
              + bo1[...])
    out_ref[...] = (_gelu(jnp.dot(u, wo2[...],
                                  preferred_element_type=jnp.float32)
                          + bo2[...]) + f_ref[...])


def _final_mlp(sums_a, sums_b, cnts, features, Wo1, bo1, Wo2, bo2):
    nb = 2000
    grid = N // nb
    full = lambda shape: pl.BlockSpec(shape, lambda i: tuple(0 for _ in shape))
    return pl.pallas_call(
        _final_body,
        grid=(grid,),
        in_specs=[
            pl.BlockSpec((NC, nb, D), lambda i: (0, i, 0)),
            pl.BlockSpec((NC, nb, D), lambda i: (0, i, 0)),
            pl.BlockSpec((NC, nb, 1), lambda i: (0, i, 0)),
            pl.BlockSpec((nb, D), lambda i: (i, 0)),
            full((D, D)), full((1, D)), full((D, D)), full((1, D)),
        ],
        out_specs=pl.BlockSpec((nb, D), lambda i: (i, 0)),
        out_shape=jax.ShapeDtypeStruct((N, D), jnp.float32),
    )(sums_a, sums_b, cnts, features, Wo1, bo1, Wo2, bo2)


def kernel(features, points, l0_edges, Wa1, ba1, Wa2, ba2,
           We1, be1, We2, be2, Wo1, bo1, Wo2, bo2):
    src = l0_edges[:, 0]
    dst = l0_edges[:, 1]

    points8 = jnp.pad(points, ((0, 0), (0, 5)))
    Wa2p = jnp.pad(Wa2, ((0, 0), (0, 5)))
    ba2p = jnp.pad(ba2, ((0, 5))).reshape(1, 8)
    Wea = We1[:D]
    Web = We1[D:2 * D]
    Wec = jnp.pad(We1[2 * D:], ((0, 5), (0, 0)))

    S, T = _node_precompute(features, points8, Wa1, ba1.reshape(1, 64),
                            Wa2p, ba2p, Wea, Web, Wec)
    zeros_d = jnp.zeros((N, D), jnp.float32)
    pat = jnp.repeat(jnp.eye(8, dtype=jnp.float32), 16, axis=1)
    EA = 128000
    srca, dsta = src[:EA], dst[:EA]
    srcb, dstb = src[EA:], dst[EA:]
    gsa, gta, cnta = _sc_gather(S, T, srca, dsta, zeros_d, pat)
    gsb, gtb, cntb = _sc_gather(S, T, srcb, dstb, zeros_d, pat)
    be1r, be2r = be1.reshape(1, D), be2.reshape(1, D)
    h2a = _edge_mlp(gsa, gta, be1r, We2, be2r)
    h2b = _edge_mlp(gsb, gtb, be1r, We2, be2r)
    (sums_a,) = _sc_scatter(h2a, dsta, zeros_d)
    (sums_b,) = _sc_scatter(h2b, dstb, zeros_d)
    cnts_packed = cnta + cntb
    cnts = cnts_packed[:, :N // 8, :].reshape(NC, N // 8, 8, 16)[..., 0]
    cnts = cnts.reshape(NC, N, 1)
    return _final_mlp(sums_a, sums_b, cnts, features, Wo1, bo1.reshape(1, D),
                      Wo2, bo2.reshape(1, D))
